# baseline (speedup 1.0000x reference)
import functools

import jax
import jax.numpy as jnp
from jax import lax
from jax.experimental import pallas as pl
from jax.experimental.pallas import tpu as pltpu
from jax.experimental.pallas import tpu_sc as plsc

_CUBE = 24
_N = _CUBE ** 3
_B = 64
_IN = 512
_OUT = 128
_STEPS = 6

_PAD = 608
_NPAD = _N + 2 * _PAD
_NW = 32
_PER_W = _N // _NW
_SUB = 144
_BAND = _SUB + 64


def _xproj_body(xT_ref, w_ref, b_ref, out_ref):
    out_ref[...] = jax.lax.dot_general(
        w_ref[...], xT_ref[...], (((1,), (0,)), ((), ())),
        preferred_element_type=jnp.float32) + b_ref[...]


def _outproj_body(h_ref, wo_ref, b_ref, out_ref):
    out_ref[...] = jax.lax.dot_general(
        h_ref[...], wo_ref[...], (((0,), (1,)), ((), ())),
        preferred_element_type=jnp.float32) + b_ref[...]


def _sc_step_body(h_ref, xp_ref, we_ref, zpad_ref, out_ref,
                  band_v, xpo_v, we_v, sem):
    c = lax.axis_index("c")
    s = lax.axis_index("s")
    wid = s * 2 + c

    for sub in range(3):
        n0 = wid * _PER_W + sub * _SUB
        cps = []
        for bi, dz in enumerate((-1, 0, 1)):
            start = n0 + dz * _CUBE * _CUBE + _PAD - 32
            cps.append(pltpu.async_copy(
                h_ref.at[pl.ds(start, _BAND)], band_v.at[bi], sem))
        cps.append(pltpu.async_copy(xp_ref.at[pl.ds(n0, _SUB)], xpo_v, sem))
        cps.append(pltpu.async_copy(we_ref.at[pl.ds(n0, _SUB)], we_v, sem))
        for cp in cps:
            cp.wait()

        @plsc.parallel_loop(0, _SUB, unroll=2)
        def body(j):
            accs = [xpo_v[j, pl.ds(16 * q, 16)] for q in range(4)]
            wv = (we_v[j, pl.ds(0, 16)], we_v[j, pl.ds(16, 16)])
            k = 0
            for dz in (-1, 0, 1):
                for dy in (-1, 0, 1):
                    for dx in (-1, 0, 1):
                        w = wv[k // 16][k % 16]
                        r = j + 32 + dy * _CUBE + dx
                        for q in range(4):
                            accs[q] = accs[q] + w * band_v[dz + 1, r,
                                                           pl.ds(16 * q, 16)]
                        k += 1
            for q in range(4):
                a = accs[q]
                e = jnp.exp(-2.0 * jnp.abs(a))
                t = (1.0 - e) / (1.0 + e)
                xpo_v[j, pl.ds(16 * q, 16)] = jnp.where(a < 0.0, -t, t)
        pltpu.sync_copy(xpo_v, out_ref.at[pl.ds(_PAD + n0, _SUB)])

    @pl.when(wid == 0)
    def _():
        pltpu.sync_copy(zpad_ref.at[pl.ds(0, _SUB)], xpo_v)
        for o in range(0, _PAD, _SUB):
            w = min(_SUB, _PAD - o)
            pltpu.sync_copy(xpo_v.at[pl.ds(0, w)], out_ref.at[pl.ds(o, w)])

    @pl.when(wid == _NW - 1)
    def _():
        pltpu.sync_copy(zpad_ref.at[pl.ds(0, _SUB)], xpo_v)
        for o in range(0, _PAD, _SUB):
            w = min(_SUB, _PAD - o)
            pltpu.sync_copy(xpo_v.at[pl.ds(0, w)],
                            out_ref.at[pl.ds(_PAD + _N + o, w)])


_sc_step = functools.partial(
    pl.kernel,
    mesh=plsc.VectorSubcoreMesh(core_axis_name="c", subcore_axis_name="s"),
    out_type=jax.ShapeDtypeStruct((_NPAD, _B), jnp.float32),
    scratch_types=[
        pltpu.VMEM((3, _BAND, _B), jnp.float32),
        pltpu.VMEM((_SUB, _B), jnp.float32),
        pltpu.VMEM((_SUB, 32), jnp.float32),
        pltpu.SemaphoreType.DMA,
    ],
)(_sc_step_body)


@jax.jit
def kernel(x, W_in_w, W_in_b, W_local, W_out_w, W_out_b, neighbor_idx):
    we = jnp.where(neighbor_idx != _N, W_local, 0.0)
    we = jnp.pad(we, ((0, 0), (0, 5)))
    xT = x.T
    b_in = W_in_b[:, None]
    b_out = W_out_b[None, :]

    grid_t = 12
    g = _N // grid_t
    xp = pl.pallas_call(
        _xproj_body,
        grid=(grid_t,),
        in_specs=[
            pl.BlockSpec((_IN, _B), lambda i: (0, 0)),
            pl.BlockSpec((g, _IN), lambda i: (i, 0)),
            pl.BlockSpec((g, 1), lambda i: (i, 0)),
        ],
        out_specs=pl.BlockSpec((g, _B), lambda i: (i, 0)),
        out_shape=jax.ShapeDtypeStruct((_N, _B), jnp.float32),
    )(xT, W_in_w, b_in)

    h = jnp.zeros((_NPAD, _B), jnp.float32)
    zpad = jnp.zeros((_PAD, _B), jnp.float32)
    for _ in range(_STEPS):
        h = _sc_step(h, xp, we, zpad)

    out = pl.pallas_call(
        _outproj_body,
        out_shape=jax.ShapeDtypeStruct((_B, _OUT), jnp.float32),
    )(h[_PAD:_PAD + _N], W_out_w, b_out)
    return out


# speedup vs baseline: 1.0656x; 1.0656x over previous
import functools

import jax
import jax.numpy as jnp
from jax import lax
from jax.experimental import pallas as pl
from jax.experimental.pallas import tpu as pltpu
from jax.experimental.pallas import tpu_sc as plsc

_CUBE = 24
_N = _CUBE ** 3
_B = 64
_IN = 512
_OUT = 128
_STEPS = 6

_PAD = 608
_NPAD = _N + 2 * _PAD
_NW = 32
_PER_W = _N // _NW
_SUB = 144
_BAND = _SUB + 64


def _xproj_body(xT_ref, w_ref, b_ref, out_ref):
    out_ref[...] = jax.lax.dot_general(
        w_ref[...], xT_ref[...], (((1,), (0,)), ((), ())),
        preferred_element_type=jnp.float32) + b_ref[...]


def _outproj_body(h_ref, wo_ref, b_ref, out_ref):
    out_ref[...] = jax.lax.dot_general(
        h_ref[...], wo_ref[...], (((0,), (1,)), ((), ())),
        preferred_element_type=jnp.float32) + b_ref[...]


def _sc_step_body(h_ref, xp_ref, we_ref, zpad_ref, out_ref,
                  band_v, xpo_v, we_v, sem):
    c = lax.axis_index("c")
    s = lax.axis_index("s")
    wid = s * 2 + c

    for sub in range(3):
        n0 = wid * _PER_W + sub * _SUB
        cps = []
        for bi, dz in enumerate((-1, 0, 1)):
            start = n0 + dz * _CUBE * _CUBE + _PAD - 32
            cps.append(pltpu.async_copy(
                h_ref.at[pl.ds(start, _BAND)], band_v.at[bi], sem))
        cps.append(pltpu.async_copy(xp_ref.at[pl.ds(n0, _SUB)], xpo_v, sem))
        cps.append(pltpu.async_copy(we_ref.at[pl.ds(n0, _SUB)], we_v, sem))
        for cp in cps:
            cp.wait()

        @plsc.parallel_loop(0, _SUB, 4)
        def body(j):
            accs = [[xpo_v[j + i, pl.ds(16 * q, 16)] for q in range(4)]
                    for i in range(4)]
            wvs = [(we_v[j + i, pl.ds(0, 16)], we_v[j + i, pl.ds(16, 16)])
                   for i in range(4)]
            for dz in (-1, 0, 1):
                for dy in (-1, 0, 1):
                    r0 = j + 31 + dy * _CUBE
                    rows = [[band_v[dz + 1, r0 + m, pl.ds(16 * q, 16)]
                             for q in range(4)] for m in range(6)]
                    for i in range(4):
                        for di in range(3):
                            k = (dz + 1) * 9 + (dy + 1) * 3 + di
                            w = wvs[i][k // 16][k % 16]
                            for q in range(4):
                                accs[i][q] = accs[i][q] + w * rows[i + di][q]
            for i in range(4):
                for q in range(4):
                    a = accs[i][q]
                    e = jnp.exp(-2.0 * jnp.abs(a))
                    t = (1.0 - e) / (1.0 + e)
                    xpo_v[j + i, pl.ds(16 * q, 16)] = jnp.where(a < 0.0, -t, t)
        pltpu.sync_copy(xpo_v, out_ref.at[pl.ds(_PAD + n0, _SUB)])

    @pl.when(wid == 0)
    def _():
        pltpu.sync_copy(zpad_ref.at[pl.ds(0, _SUB)], xpo_v)
        for o in range(0, _PAD, _SUB):
            w = min(_SUB, _PAD - o)
            pltpu.sync_copy(xpo_v.at[pl.ds(0, w)], out_ref.at[pl.ds(o, w)])

    @pl.when(wid == _NW - 1)
    def _():
        pltpu.sync_copy(zpad_ref.at[pl.ds(0, _SUB)], xpo_v)
        for o in range(0, _PAD, _SUB):
            w = min(_SUB, _PAD - o)
            pltpu.sync_copy(xpo_v.at[pl.ds(0, w)],
                            out_ref.at[pl.ds(_PAD + _N + o, w)])


_sc_step = functools.partial(
    pl.kernel,
    mesh=plsc.VectorSubcoreMesh(core_axis_name="c", subcore_axis_name="s"),
    out_type=jax.ShapeDtypeStruct((_NPAD, _B), jnp.float32),
    scratch_types=[
        pltpu.VMEM((3, _BAND, _B), jnp.float32),
        pltpu.VMEM((_SUB, _B), jnp.float32),
        pltpu.VMEM((_SUB, 32), jnp.float32),
        pltpu.SemaphoreType.DMA,
    ],
)(_sc_step_body)


@jax.jit
def kernel(x, W_in_w, W_in_b, W_local, W_out_w, W_out_b, neighbor_idx):
    we = jnp.where(neighbor_idx != _N, W_local, 0.0)
    we = jnp.pad(we, ((0, 0), (0, 5)))
    xT = x.T
    b_in = W_in_b[:, None]
    b_out = W_out_b[None, :]

    grid_t = 12
    g = _N // grid_t
    xp = pl.pallas_call(
        _xproj_body,
        grid=(grid_t,),
        in_specs=[
            pl.BlockSpec((_IN, _B), lambda i: (0, 0)),
            pl.BlockSpec((g, _IN), lambda i: (i, 0)),
            pl.BlockSpec((g, 1), lambda i: (i, 0)),
        ],
        out_specs=pl.BlockSpec((g, _B), lambda i: (i, 0)),
        out_shape=jax.ShapeDtypeStruct((_N, _B), jnp.float32),
    )(xT, W_in_w, b_in)

    h = jnp.zeros((_NPAD, _B), jnp.float32)
    zpad = jnp.zeros((_PAD, _B), jnp.float32)
    for _ in range(_STEPS):
        h = _sc_step(h, xp, we, zpad)

    out = pl.pallas_call(
        _outproj_body,
        out_shape=jax.ShapeDtypeStruct((_B, _OUT), jnp.float32),
    )(h[_PAD:_PAD + _N], W_out_w, b_out)
    return out
